# baseline (speedup 1.0000x reference)
import jax
import jax.numpy as jnp
from jax.experimental import pallas as pl
from jax.experimental.pallas import tpu as pltpu

_N = 4
_P = 4096
_BP = 2048
_R = _P // _BP


def _chamfer_kernel(p1_ref, p2_ref, out_ref, colmin_ref, acc_ref, b2_ref):
    n = pl.program_id(0)
    r = pl.program_id(1)

    a = p1_ref[0]
    b = p2_ref[0]

    ab = jax.lax.dot_general(
        a, b, (((1,), (1,)), ((), ())), preferred_element_type=jnp.float32
    )

    a2 = jnp.sum(a * a, axis=1, keepdims=True)

    @pl.when(r == 0)
    def _build_b2():
        b2_ref[...] = jnp.sum(b * b, axis=1, keepdims=True).T

    row_red = jnp.min(b2_ref[...] - 2.0 * ab, axis=1, keepdims=True)
    row_sum = jnp.sum(row_red + a2)

    col_red = jnp.min(a2 - 2.0 * ab, axis=0, keepdims=True)

    @pl.when(jnp.logical_and(n == 0, r == 0))
    def _init_acc():
        acc_ref[...] = jnp.zeros((1, 1), jnp.float32)

    @pl.when(r == 0)
    def _init_colmin():
        colmin_ref[...] = col_red

    @pl.when(r > 0)
    def _merge_colmin():
        colmin_ref[...] = jnp.minimum(colmin_ref[...], col_red)

    acc_ref[...] += row_sum[None, None]

    @pl.when(r == _R - 1)
    def _fold_colmin():
        acc_ref[...] += jnp.sum(colmin_ref[...] + b2_ref[...])[None, None]

    @pl.when(jnp.logical_and(n == _N - 1, r == _R - 1))
    def _finalize():
        out_ref[...] = acc_ref[...] * (1.0 / (_N * _P))


@jax.jit
def kernel(p1, p2):
    p1p = jnp.pad(p1, ((0, 0), (0, 0), (0, 5)))
    p2p = jnp.pad(p2, ((0, 0), (0, 0), (0, 5)))

    out = pl.pallas_call(
        _chamfer_kernel,
        grid=(_N, _R),
        in_specs=[
            pl.BlockSpec((1, _BP, 8), lambda n, r: (n, r, 0)),
            pl.BlockSpec((1, _P, 8), lambda n, r: (n, 0, 0)),
        ],
        out_specs=pl.BlockSpec((1, 1), lambda n, r: (0, 0)),
        out_shape=jax.ShapeDtypeStruct((1, 1), jnp.float32),
        scratch_shapes=[
            pltpu.VMEM((1, _P), jnp.float32),
            pltpu.VMEM((1, 1), jnp.float32),
            pltpu.VMEM((1, _P), jnp.float32),
        ],
        compiler_params=pltpu.CompilerParams(
            vmem_limit_bytes=100 * 1024 * 1024,
        ),
    )(p1p, p2p)
    return out[0, 0]


# speedup vs baseline: 1.4838x; 1.4838x over previous
import jax
import jax.numpy as jnp
from jax.experimental import pallas as pl
from jax.experimental.pallas import tpu as pltpu

_N = 4
_P = 4096
_BC = 512
_NC = _P // _BC
_L = 128


def _chamfer_kernel(p1_ref, p2_ref, out_ref):
    tot = jnp.zeros((1, 1), jnp.float32)
    for n in range(_N):
        a = p1_ref[n]
        b = p2_ref[n]
        a2 = jnp.sum(a * a, axis=1, keepdims=True)
        b2r = jnp.sum(b * b, axis=1, keepdims=True).T

        rowm = None
        for c in range(_NC):
            bc = -2.0 * b[c * _BC:(c + 1) * _BC, :]
            ab = jax.lax.dot_general(
                a, bc, (((1,), (1,)), ((), ())),
                preferred_element_type=jnp.float32,
            )
            for k in range(_BC // _L):
                j0 = c * _BC + k * _L
                t = b2r[:, j0:j0 + _L] + ab[:, k * _L:(k + 1) * _L]
                rowm = t if rowm is None else jnp.minimum(rowm, t)
            colc = jnp.min(a2 + ab, axis=0, keepdims=True)
            tot += jnp.sum(colc + b2r[:, c * _BC:(c + 1) * _BC])[None, None]

        rowfin = jnp.min(rowm, axis=1, keepdims=True)
        tot += jnp.sum(rowfin + a2)[None, None]

    out_ref[...] = tot * (1.0 / (_N * _P))


@jax.jit
def kernel(p1, p2):
    p1p = jnp.pad(p1, ((0, 0), (0, 0), (0, 5)))
    p2p = jnp.pad(p2, ((0, 0), (0, 0), (0, 5)))

    out = pl.pallas_call(
        _chamfer_kernel,
        out_shape=jax.ShapeDtypeStruct((1, 1), jnp.float32),
        compiler_params=pltpu.CompilerParams(
            vmem_limit_bytes=100 * 1024 * 1024,
        ),
    )(p1p, p2p)
    return out[0, 0]


# speedup vs baseline: 1.5161x; 1.0218x over previous
import jax
import jax.numpy as jnp
from jax.experimental import pallas as pl
from jax.experimental.pallas import tpu as pltpu

_N = 4
_P = 4096
_BC = 512
_NC = _P // _BC
_L = 128


def _chamfer_kernel(p1_ref, p2_ref, out_ref, acc_ref):
    n = pl.program_id(0)

    a = p1_ref[0]
    b = p2_ref[0]
    a2 = jnp.sum(a * a, axis=1, keepdims=True)
    b2r = jnp.sum(b * b, axis=1, keepdims=True).T

    tot = jnp.zeros((1, 1), jnp.float32)
    rowm = None
    for c in range(_NC):
        bc = -2.0 * b[c * _BC:(c + 1) * _BC, :]
        ab = jax.lax.dot_general(
            a, bc, (((1,), (1,)), ((), ())),
            preferred_element_type=jnp.float32,
        )
        for k in range(_BC // _L):
            j0 = c * _BC + k * _L
            t = b2r[:, j0:j0 + _L] + ab[:, k * _L:(k + 1) * _L]
            rowm = t if rowm is None else jnp.minimum(rowm, t)
        colc = jnp.min(a2 + ab, axis=0, keepdims=True)
        tot += jnp.sum(colc + b2r[:, c * _BC:(c + 1) * _BC])[None, None]

    rowfin = jnp.min(rowm, axis=1, keepdims=True)
    tot += jnp.sum(rowfin + a2)[None, None]

    @pl.when(n == 0)
    def _init():
        acc_ref[...] = jnp.zeros((1, 1), jnp.float32)

    acc_ref[...] += tot

    @pl.when(n == _N - 1)
    def _fin():
        out_ref[...] = acc_ref[...] * (1.0 / (_N * _P))


@jax.jit
def kernel(p1, p2):
    p1p = jnp.pad(p1, ((0, 0), (0, 0), (0, 5)))
    p2p = jnp.pad(p2, ((0, 0), (0, 0), (0, 5)))

    out = pl.pallas_call(
        _chamfer_kernel,
        grid=(_N,),
        in_specs=[
            pl.BlockSpec((1, _P, 8), lambda n: (n, 0, 0)),
            pl.BlockSpec((1, _P, 8), lambda n: (n, 0, 0)),
        ],
        out_specs=pl.BlockSpec((1, 1), lambda n: (0, 0)),
        out_shape=jax.ShapeDtypeStruct((1, 1), jnp.float32),
        scratch_shapes=[
            pltpu.VMEM((1, 1), jnp.float32),
        ],
        compiler_params=pltpu.CompilerParams(
            vmem_limit_bytes=100 * 1024 * 1024,
        ),
    )(p1p, p2p)
    return out[0, 0]


# speedup vs baseline: 1.8673x; 1.2316x over previous
import jax
import jax.numpy as jnp
from jax.experimental import pallas as pl
from jax.experimental.pallas import tpu as pltpu

_N = 4
_P = 4096
_BC = 512
_NC = _P // _BC
_L = 128


def _chamfer_kernel(p1_ref, p2_ref, out_ref, acc_ref):
    n = pl.program_id(0)

    a = p1_ref[0]
    b = p2_ref[0]
    a2 = jnp.sum(a * a, axis=1, keepdims=True)
    b2r = jnp.sum(b * b, axis=1, keepdims=True).T

    tot = jnp.zeros((1, 1), jnp.float32)
    rowm = None
    for c in range(_NC):
        bc = -2.0 * b[c * _BC:(c + 1) * _BC, :]
        ab = jax.lax.dot_general(
            a, bc, (((1,), (1,)), ((), ())),
            preferred_element_type=jnp.float32,
        )
        for k in range(_BC // _L):
            j0 = c * _BC + k * _L
            t = b2r[:, j0:j0 + _L] + ab[:, k * _L:(k + 1) * _L]
            rowm = t if rowm is None else jnp.minimum(rowm, t)
        colc = jnp.min(a2 + ab, axis=0, keepdims=True)
        tot += jnp.sum(colc + b2r[:, c * _BC:(c + 1) * _BC])[None, None]

    rowfin = jnp.min(rowm, axis=1, keepdims=True)
    tot += jnp.sum(rowfin + a2)[None, None]

    @pl.when(n == 0)
    def _init():
        acc_ref[...] = jnp.zeros((1, 1), jnp.float32)

    acc_ref[...] += tot

    @pl.when(n == _N - 1)
    def _fin():
        out_ref[...] = acc_ref[...] * (1.0 / (_N * _P))


@jax.jit
def kernel(p1, p2):
    p1p = p1
    p2p = p2

    out = pl.pallas_call(
        _chamfer_kernel,
        grid=(_N,),
        in_specs=[
            pl.BlockSpec((1, _P, 3), lambda n: (n, 0, 0)),
            pl.BlockSpec((1, _P, 3), lambda n: (n, 0, 0)),
        ],
        out_specs=pl.BlockSpec((1, 1), lambda n: (0, 0)),
        out_shape=jax.ShapeDtypeStruct((1, 1), jnp.float32),
        scratch_shapes=[
            pltpu.VMEM((1, 1), jnp.float32),
        ],
        compiler_params=pltpu.CompilerParams(
            vmem_limit_bytes=100 * 1024 * 1024,
        ),
    )(p1p, p2p)
    return out[0, 0]
